# baseline (speedup 1.0000x reference)
import jax
import jax.numpy as jnp
from jax import lax
from jax.experimental import pallas as pl
from jax.experimental.pallas import tpu as pltpu
from jax.experimental.pallas import tpu_sc as plsc

B, H, D = 8, 16, 128
MAX_CACHE = 2048
NUM_TOK = 1024

_NC, _NS = 2, 16
_NW = _NC * _NS
_BH = B * H
_PLANES_PER_W = _BH // _NW
_CHUNK = 128
_CHUNKS_PER_PLANE = NUM_TOK // _CHUNK
_NCHUNK = _PLANES_PER_W * _CHUNKS_PER_PLANE
_NBUF = 4
_POS_VECS = NUM_TOK // 16
_TBLK = 512


def _sc_body(ip_hbm, v_hbm, pos_hbm, vo_hbm, po_hbm,
             ip_v, idx_v, vb, posb, sgv, ssv):
    wid = lax.axis_index("s") * _NC + lax.axis_index("c")

    pltpu.sync_copy(ip_hbm, ip_v)

    @pl.when(wid < B)
    def _():
        pltpu.sync_copy(pos_hbm.at[wid], posb)
        for j in range(_POS_VECS):
            vals = ip_v[pl.ds(j * 16, 16)]
            plsc.store_scatter(posb, [vals], vals)
        pltpu.sync_copy(posb, po_hbm.at[wid])

    def issue_gather(slot, c):
        plane = c // _CHUNKS_PER_PLANE
        t0 = (c % _CHUNKS_PER_PLANE) * _CHUNK
        base = (wid * _PLANES_PER_W + plane) * NUM_TOK
        pltpu.async_copy(v_hbm.at[pl.ds(base + t0, _CHUNK)], vb.at[slot], sgv)
        for j in range(_CHUNK // 16):
            idx_v[slot, pl.ds(j * 16, 16)] = ip_v[pl.ds(t0 + j * 16, 16)] + base

    def drain(sem):
        pltpu.make_async_copy(v_hbm.at[pl.ds(0, _CHUNK)], vb.at[0], sem).wait()

    issue_gather(0, 0)
    issue_gather(1, 1)

    def outer(i, carry):
        for slot in range(_NBUF):
            c = i * _NBUF + slot

            @pl.when(c >= 2)
            def _():
                drain(ssv)

            drain(sgv)
            pltpu.async_copy(vb.at[slot], vo_hbm.at[idx_v.at[slot]], ssv)

            nslot = (slot + 2) % _NBUF

            @pl.when(c + 2 < _NCHUNK)
            def _():
                issue_gather(nslot, c + 2)
        return carry

    lax.fori_loop(0, _NCHUNK // _NBUF, outer, 0)
    drain(ssv)
    drain(ssv)


def _tc_body(pos_sref, k_ref, ko_ref):
    del pos_sref
    ko_ref[...] = k_ref[...]


@jax.jit
def _kv_update(input_pos, k_src, v_src, pos_src):
    n_rows = _BH * NUM_TOK
    sc_run = pl.kernel(
        _sc_body,
        out_type=(
            jax.ShapeDtypeStruct((n_rows, D), jnp.float32),
            jax.ShapeDtypeStruct((B, MAX_CACHE), jnp.int32),
        ),
        mesh=plsc.VectorSubcoreMesh(core_axis_name="c", subcore_axis_name="s"),
        compiler_params=pltpu.CompilerParams(needs_layout_passes=False),
        scratch_types=(
            pltpu.VMEM((NUM_TOK,), jnp.int32),
            pltpu.VMEM((_NBUF, _CHUNK), jnp.int32),
            pltpu.VMEM((_NBUF, _CHUNK, D), jnp.float32),
            pltpu.VMEM((MAX_CACHE,), jnp.int32),
            pltpu.SemaphoreType.DMA,
            pltpu.SemaphoreType.DMA,
        ),
    )
    v_flat, pos_out = sc_run(input_pos, v_src, pos_src)

    k_out = pl.pallas_call(
        _tc_body,
        grid_spec=pltpu.PrefetchScalarGridSpec(
            num_scalar_prefetch=1,
            grid=(_BH, NUM_TOK // _TBLK),
            in_specs=[
                pl.BlockSpec((1, _TBLK, D), lambda bh, t, pos: (bh, t, 0)),
            ],
            out_specs=pl.BlockSpec(
                (1, _TBLK, D),
                lambda bh, t, pos: (bh, pos[t * _TBLK] // _TBLK, 0),
            ),
        ),
        out_shape=jax.ShapeDtypeStruct((_BH, NUM_TOK, D), jnp.float32),
        compiler_params=pltpu.CompilerParams(
            dimension_semantics=("arbitrary", "arbitrary"),
        ),
    )(input_pos, k_src)
    return k_out, v_flat, pos_out


def kernel(input_pos, k_val, v_val, k_cache, v_cache, pos):
    del k_cache, v_cache
    k_src = k_val.reshape(_BH, NUM_TOK, D)
    v_src = v_val.reshape(_BH * NUM_TOK, D)
    pos_src = pos.reshape(B, MAX_CACHE)
    k_out, v_flat, pos_flat = _kv_update(input_pos, k_src, v_src, pos_src)
    return (
        k_out.reshape(B, H, NUM_TOK, D),
        v_flat.reshape(B, H, NUM_TOK, D),
        pos_flat.reshape(B, 1, MAX_CACHE),
    )


# speedup vs baseline: 1.0005x; 1.0005x over previous
import jax
import jax.numpy as jnp
from jax import lax
from jax.experimental import pallas as pl
from jax.experimental.pallas import tpu as pltpu
from jax.experimental.pallas import tpu_sc as plsc

B, H, D = 8, 16, 128
MAX_CACHE = 2048
NUM_TOK = 1024

_NC, _NS = 2, 16
_NW = _NC * _NS
_BH = B * H
_PLANES_PER_W = _BH // _NW
_CHUNK = 128
_CHUNKS_PER_PLANE = NUM_TOK // _CHUNK
_NCHUNK = _PLANES_PER_W * _CHUNKS_PER_PLANE
_NBUF = 4
_POS_VECS = NUM_TOK // 16
_TBLK = 512


def _sc_body(ip_hbm, v_hbm, pos_hbm, vo_hbm, po_hbm,
             ip_v, idx_v, vb, posb, sgv, ssv):
    wid = lax.axis_index("s") * _NC + lax.axis_index("c")

    pltpu.sync_copy(ip_hbm, ip_v)

    @pl.when(wid < B)
    def _():
        pltpu.sync_copy(pos_hbm.at[wid], posb)
        for j in range(_POS_VECS):
            vals = ip_v[pl.ds(j * 16, 16)]
            plsc.store_scatter(posb, [vals], vals)
        pltpu.sync_copy(posb, po_hbm.at[wid])

    def issue_gather(slot, c):
        plane = c // _CHUNKS_PER_PLANE
        t0 = (c % _CHUNKS_PER_PLANE) * _CHUNK
        base = (wid * _PLANES_PER_W + plane) * NUM_TOK
        pltpu.async_copy(v_hbm.at[pl.ds(base + t0, _CHUNK)], vb.at[slot], sgv)
        for j in range(_CHUNK // 16):
            idx_v[slot, pl.ds(j * 16, 16)] = ip_v[pl.ds(t0 + j * 16, 16)] + base

    def drain(sem):
        pltpu.make_async_copy(v_hbm.at[pl.ds(0, _CHUNK)], vb.at[0], sem).wait()

    issue_gather(0, 0)
    issue_gather(1, 1)

    def outer(i, carry):
        for slot in range(_NBUF):
            c = i * _NBUF + slot

            @pl.when(c >= 2)
            def _():
                drain(ssv)

            drain(sgv)
            pltpu.async_copy(vb.at[slot], vo_hbm.at[idx_v.at[slot]], ssv)

            nslot = (slot + 2) % _NBUF

            @pl.when(c + 2 < _NCHUNK)
            def _():
                issue_gather(nslot, c + 2)
        return carry

    lax.fori_loop(0, _NCHUNK // _NBUF, outer, 0)
    drain(ssv)
    drain(ssv)


def _tc_body(pos_sref, k_ref, ko_ref):
    del pos_sref
    ko_ref[...] = k_ref[...]


@jax.jit
def _kv_update(input_pos, k_src, v_src, pos_src):
    n_rows = _BH * NUM_TOK
    sc_run = pl.kernel(
        _sc_body,
        out_type=(
            jax.ShapeDtypeStruct((n_rows, D), jnp.float32),
            jax.ShapeDtypeStruct((B, MAX_CACHE), jnp.int32),
        ),
        mesh=plsc.VectorSubcoreMesh(core_axis_name="c", subcore_axis_name="s"),
        compiler_params=pltpu.CompilerParams(needs_layout_passes=False),
        scratch_types=(
            pltpu.VMEM((NUM_TOK,), jnp.int32),
            pltpu.VMEM((_NBUF, _CHUNK), jnp.int32),
            pltpu.VMEM((_NBUF, _CHUNK, D), jnp.float32),
            pltpu.VMEM((MAX_CACHE,), jnp.int32),
            pltpu.SemaphoreType.DMA,
            pltpu.SemaphoreType.DMA,
        ),
    )
    k_out = pl.pallas_call(
        _tc_body,
        grid_spec=pltpu.PrefetchScalarGridSpec(
            num_scalar_prefetch=1,
            grid=(_BH, NUM_TOK // _TBLK),
            in_specs=[
                pl.BlockSpec((1, _TBLK, D), lambda bh, t, pos: (bh, t, 0)),
            ],
            out_specs=pl.BlockSpec(
                (1, _TBLK, D),
                lambda bh, t, pos: (bh, pos[t * _TBLK] // _TBLK, 0),
            ),
        ),
        out_shape=jax.ShapeDtypeStruct((_BH, NUM_TOK, D), jnp.float32),
        compiler_params=pltpu.CompilerParams(
            dimension_semantics=("arbitrary", "arbitrary"),
        ),
    )(input_pos, k_src)
    v_flat, pos_out = sc_run(input_pos, v_src, pos_src)
    return k_out, v_flat, pos_out


def kernel(input_pos, k_val, v_val, k_cache, v_cache, pos):
    del k_cache, v_cache
    k_src = k_val.reshape(_BH, NUM_TOK, D)
    v_src = v_val.reshape(_BH * NUM_TOK, D)
    pos_src = pos.reshape(B, MAX_CACHE)
    k_out, v_flat, pos_flat = _kv_update(input_pos, k_src, v_src, pos_src)
    return (
        k_out.reshape(B, H, NUM_TOK, D),
        v_flat.reshape(B, H, NUM_TOK, D),
        pos_flat.reshape(B, 1, MAX_CACHE),
    )


# speedup vs baseline: 1.7742x; 1.7734x over previous
import jax
import jax.numpy as jnp
from jax import lax
from jax.experimental import pallas as pl
from jax.experimental.pallas import tpu as pltpu
from jax.experimental.pallas import tpu_sc as plsc

B, H, D = 8, 16, 128
MAX_CACHE = 2048
NUM_TOK = 1024

_NC, _NS = 2, 16
_NW = _NC * _NS
_BH = B * H
_PLANES_PER_W = _BH // _NW
_CHUNK = 128
_CHUNKS_PER_PLANE = NUM_TOK // _CHUNK
_NTASK = 2 * _PLANES_PER_W * _CHUNKS_PER_PLANE
_NBUF = 4
_POS_SEG = MAX_CACHE // (_NW // B)
_POS_VECS = NUM_TOK // 16


def _sc_body(ip_hbm, k_hbm, v_hbm, pos_hbm,
             ko_hbm, vo_hbm, po_hbm,
             ip_v, idx_v, buf, posb, sg, ss):
    wid = lax.axis_index("s") * _NC + lax.axis_index("c")

    pltpu.sync_copy(ip_hbm, ip_v)

    bq = wid // (_NW // B)
    seg = (wid % (_NW // B)) * _POS_SEG
    pltpu.sync_copy(pos_hbm.at[bq, pl.ds(seg, _POS_SEG)], posb)
    for j in range(_POS_VECS):
        vals = ip_v[pl.ds(j * 16, 16)]
        mask = (vals >= seg) & (vals < seg + _POS_SEG)
        plsc.store_scatter(posb, [vals - seg], vals, mask=mask)
    pltpu.sync_copy(posb, po_hbm.at[bq, pl.ds(seg, _POS_SEG)])

    def issue_gather(slot, t, src_hbm):
        c = t // 2
        plane = c // _CHUNKS_PER_PLANE
        t0 = (c % _CHUNKS_PER_PLANE) * _CHUNK
        base = (wid * _PLANES_PER_W + plane) * NUM_TOK
        pltpu.async_copy(src_hbm.at[pl.ds(base + t0, _CHUNK)], buf.at[slot], sg)
        for j in range(_CHUNK // 16):
            idx_v[slot, pl.ds(j * 16, 16)] = ip_v[pl.ds(t0 + j * 16, 16)] + base

    def drain(sem):
        pltpu.make_async_copy(k_hbm.at[pl.ds(0, _CHUNK)], buf.at[0], sem).wait()

    issue_gather(0, 0, k_hbm)
    issue_gather(1, 1, v_hbm)

    def outer(i, carry):
        for slot in range(_NBUF):
            t = i * _NBUF + slot
            dst_hbm = ko_hbm if slot % 2 == 0 else vo_hbm

            @pl.when(t >= 2)
            def _():
                drain(ss)

            drain(sg)
            pltpu.async_copy(buf.at[slot], dst_hbm.at[idx_v.at[slot]], ss)

            nslot = (slot + 2) % _NBUF
            nsrc = k_hbm if nslot % 2 == 0 else v_hbm

            @pl.when(t + 2 < _NTASK)
            def _():
                issue_gather(nslot, t + 2, nsrc)
        return carry

    lax.fori_loop(0, _NTASK // _NBUF, outer, 0)
    drain(ss)
    drain(ss)


@jax.jit
def _sc_update(input_pos, k_src, v_src, pos_src):
    n_rows = _BH * NUM_TOK
    run = pl.kernel(
        _sc_body,
        out_type=(
            jax.ShapeDtypeStruct((n_rows, D), jnp.float32),
            jax.ShapeDtypeStruct((n_rows, D), jnp.float32),
            jax.ShapeDtypeStruct((B, MAX_CACHE), jnp.int32),
        ),
        mesh=plsc.VectorSubcoreMesh(core_axis_name="c", subcore_axis_name="s"),
        compiler_params=pltpu.CompilerParams(needs_layout_passes=False),
        scratch_types=(
            pltpu.VMEM((NUM_TOK,), jnp.int32),
            pltpu.VMEM((_NBUF, _CHUNK), jnp.int32),
            pltpu.VMEM((_NBUF, _CHUNK, D), jnp.float32),
            pltpu.VMEM((_POS_SEG,), jnp.int32),
            pltpu.SemaphoreType.DMA,
            pltpu.SemaphoreType.DMA,
        ),
    )
    return run(input_pos, k_src, v_src, pos_src)


def kernel(input_pos, k_val, v_val, k_cache, v_cache, pos):
    del k_cache, v_cache
    k_src = k_val.reshape(_BH * NUM_TOK, D)
    v_src = v_val.reshape(_BH * NUM_TOK, D)
    pos_src = pos.reshape(B, MAX_CACHE)
    k_flat, v_flat, pos_flat = _sc_update(input_pos, k_src, v_src, pos_src)
    return (
        k_flat.reshape(B, H, NUM_TOK, D),
        v_flat.reshape(B, H, NUM_TOK, D),
        pos_flat.reshape(B, 1, MAX_CACHE),
    )


# speedup vs baseline: 1.8160x; 1.0235x over previous
import jax
import jax.numpy as jnp
from jax import lax
from jax.experimental import pallas as pl
from jax.experimental.pallas import tpu as pltpu
from jax.experimental.pallas import tpu_sc as plsc

B, H, D = 8, 16, 128
MAX_CACHE = 2048
NUM_TOK = 1024

_NC, _NS = 2, 16
_NW = _NC * _NS
_BH = B * H

_NSTREAM = 28
_SPLANES = 3
_NPUMP = _NW - _NSTREAM
_PPLANES = 11
_PBASE = _NSTREAM * _SPLANES

_CHUNK = 128
_CHUNKS_PER_PLANE = NUM_TOK // _CHUNK
_NTASK_S = 2 * _SPLANES * _CHUNKS_PER_PLANE
_PCHUNK = 512
_HALVES = NUM_TOK // _PCHUNK
_NTASK_P = 2 * _PPLANES * _HALVES
_NBUF = 4

_POS_SEG = MAX_CACHE // (_NW // B)
_POS_VECS = NUM_TOK // 16


def _sc_body(ip_hbm, k_hbm, v_hbm, pos_hbm,
             ko_hbm, vo_hbm, po_hbm,
             ip_v, idx_v, buf, posb, spm, sg, ss, sg2, ss2):
    wid = lax.axis_index("s") * _NC + lax.axis_index("c")

    pltpu.sync_copy(ip_hbm, ip_v)

    bq = wid // (_NW // B)
    seg = (wid % (_NW // B)) * _POS_SEG
    pltpu.sync_copy(pos_hbm.at[bq, pl.ds(seg, _POS_SEG)], posb)
    for j in range(_POS_VECS):
        vals = ip_v[pl.ds(j * 16, 16)]
        mask = (vals >= seg) & (vals < seg + _POS_SEG)
        plsc.store_scatter(posb, [vals - seg], vals, mask=mask)
    pltpu.sync_copy(posb, po_hbm.at[bq, pl.ds(seg, _POS_SEG)])

    def s_issue_gather(slot, t, src_hbm):
        c = t // 2
        plane = wid * _SPLANES + c // _CHUNKS_PER_PLANE
        t0 = (c % _CHUNKS_PER_PLANE) * _CHUNK
        base = plane * NUM_TOK
        pltpu.async_copy(src_hbm.at[pl.ds(base + t0, _CHUNK)], buf.at[slot], sg)
        for j in range(_CHUNK // 16):
            idx_v[slot, pl.ds(j * 16, 16)] = ip_v[pl.ds(t0 + j * 16, 16)] + base

    def s_drain(sem):
        pltpu.make_async_copy(k_hbm.at[pl.ds(0, _CHUNK)], buf.at[0], sem).wait()

    @pl.when(wid < _NSTREAM)
    def _():
        s_issue_gather(0, 0, k_hbm)
        s_issue_gather(1, 1, v_hbm)

        def outer(i, carry):
            for slot in range(_NBUF):
                t = i * _NBUF + slot
                dst_hbm = ko_hbm if slot % 2 == 0 else vo_hbm

                @pl.when(t >= 2)
                def _():
                    s_drain(ss)

                s_drain(sg)
                pltpu.async_copy(buf.at[slot], dst_hbm.at[idx_v.at[slot]], ss)

                nslot = (slot + 2) % _NBUF
                nsrc = k_hbm if nslot % 2 == 0 else v_hbm

                @pl.when(t + 2 < _NTASK_S)
                def _():
                    s_issue_gather(nslot, t + 2, nsrc)
            return carry

        lax.fori_loop(0, _NTASK_S // _NBUF, outer, 0)
        s_drain(ss)
        s_drain(ss)

    pid = wid - _NSTREAM
    lslot = pid // _NC

    def p_task(t):
        c = t // 2
        plane = _PBASE + pid * _PPLANES + c // _HALVES
        t0 = (c % _HALVES) * _PCHUNK
        pos0 = jnp.min(ip_v[pl.ds(t0, 16)])
        pos0 = pl.multiple_of(pos0, _PCHUNK)
        src = plane * NUM_TOK + t0
        dst = plane * NUM_TOK + pos0
        return src, dst

    def p_issue_gather(slot, t, src_hbm):
        src, _ = p_task(t)
        pltpu.async_copy(src_hbm.at[pl.ds(src, _PCHUNK)], spm.at[lslot, slot], sg2)

    def p_drain(sem):
        pltpu.make_async_copy(
            k_hbm.at[pl.ds(0, _PCHUNK)], spm.at[0, 0], sem).wait()

    @pl.when(wid >= _NSTREAM)
    def _():
        p_issue_gather(0, 0, k_hbm)
        p_issue_gather(1, 1, v_hbm)

        def outer(i, carry):
            for slot in range(_NBUF):
                t = i * _NBUF + slot
                dst_hbm = ko_hbm if slot % 2 == 0 else vo_hbm

                @pl.when(t >= 2)
                def _():
                    p_drain(ss2)

                p_drain(sg2)
                _, dst = p_task(t)
                pltpu.async_copy(
                    spm.at[lslot, slot], dst_hbm.at[pl.ds(dst, _PCHUNK)], ss2)

                nslot = (slot + 2) % _NBUF
                nsrc = k_hbm if nslot % 2 == 0 else v_hbm

                @pl.when(t + 2 < _NTASK_P)
                def _():
                    p_issue_gather(nslot, t + 2, nsrc)
            return carry

        lax.fori_loop(0, _NTASK_P // _NBUF, outer, 0)
        p_drain(ss2)
        p_drain(ss2)


@jax.jit
def _sc_update(input_pos, k_src, v_src, pos_src):
    n_rows = _BH * NUM_TOK
    run = pl.kernel(
        _sc_body,
        out_type=(
            jax.ShapeDtypeStruct((n_rows, D), jnp.float32),
            jax.ShapeDtypeStruct((n_rows, D), jnp.float32),
            jax.ShapeDtypeStruct((B, MAX_CACHE), jnp.int32),
        ),
        mesh=plsc.VectorSubcoreMesh(core_axis_name="c", subcore_axis_name="s"),
        compiler_params=pltpu.CompilerParams(needs_layout_passes=False),
        scratch_types=(
            pltpu.VMEM((NUM_TOK,), jnp.int32),
            pltpu.VMEM((_NBUF, _CHUNK), jnp.int32),
            pltpu.VMEM((_NBUF, _CHUNK, D), jnp.float32),
            pltpu.VMEM((_POS_SEG,), jnp.int32),
            pltpu.VMEM_SHARED((2, _NBUF, _PCHUNK, D), jnp.float32),
            pltpu.SemaphoreType.DMA,
            pltpu.SemaphoreType.DMA,
            pltpu.SemaphoreType.DMA,
            pltpu.SemaphoreType.DMA,
        ),
    )
    return run(input_pos, k_src, v_src, pos_src)


def kernel(input_pos, k_val, v_val, k_cache, v_cache, pos):
    del k_cache, v_cache
    k_src = k_val.reshape(_BH * NUM_TOK, D)
    v_src = v_val.reshape(_BH * NUM_TOK, D)
    pos_src = pos.reshape(B, MAX_CACHE)
    k_flat, v_flat, pos_flat = _sc_update(input_pos, k_src, v_src, pos_src)
    return (
        k_flat.reshape(B, H, NUM_TOK, D),
        v_flat.reshape(B, H, NUM_TOK, D),
        pos_flat.reshape(B, 1, MAX_CACHE),
    )


# speedup vs baseline: 1.8304x; 1.0080x over previous
import jax
import jax.numpy as jnp
from jax import lax
from jax.experimental import pallas as pl
from jax.experimental.pallas import tpu as pltpu
from jax.experimental.pallas import tpu_sc as plsc

B, H, D = 8, 16, 128
MAX_CACHE = 2048
NUM_TOK = 1024

_NC, _NS = 2, 16
_NW = _NC * _NS
_BH = B * H

_NSTREAM = 28
_SPLANES = 3
_NPUMP = _NW - _NSTREAM
_PPLANES = 11
_PBASE = _NSTREAM * _SPLANES

_CHUNK = 128
_CHUNKS_PER_PLANE = NUM_TOK // _CHUNK
_NTASK_S = 2 * _SPLANES * _CHUNKS_PER_PLANE
_PCHUNK = 512
_HALVES = NUM_TOK // _PCHUNK
_NTASK_P = 2 * _PPLANES * _HALVES
_NBUF = 4

_POS_SEG = MAX_CACHE // (_NW // B)
_POS_VECS = NUM_TOK // 16


def _sc_body(ip_hbm, k_hbm, v_hbm, pos_hbm,
             ko_hbm, vo_hbm, po_hbm,
             ip_v, idx_v, buf, posb, spm, sg, ss, sg2, ss2):
    wid = lax.axis_index("s") * _NC + lax.axis_index("c")

    pltpu.sync_copy(ip_hbm, ip_v)

    def s_issue_gather(slot, t, src_hbm):
        c = t // 2
        plane = wid * _SPLANES + c // _CHUNKS_PER_PLANE
        t0 = (c % _CHUNKS_PER_PLANE) * _CHUNK
        base = plane * NUM_TOK
        pltpu.async_copy(src_hbm.at[pl.ds(base + t0, _CHUNK)], buf.at[slot], sg)
        for j in range(_CHUNK // 16):
            idx_v[slot, pl.ds(j * 16, 16)] = ip_v[pl.ds(t0 + j * 16, 16)] + base

    def s_drain(sem):
        pltpu.make_async_copy(k_hbm.at[pl.ds(0, _CHUNK)], buf.at[0], sem).wait()

    @pl.when(wid < _NSTREAM)
    def _():
        s_issue_gather(0, 0, k_hbm)
        s_issue_gather(1, 1, v_hbm)

    pid = wid - _NSTREAM
    lslot = pid // _NC

    def p_task(t):
        c = t // 2
        plane = _PBASE + pid * _PPLANES + c // _HALVES
        t0 = (c % _HALVES) * _PCHUNK
        pos0 = jnp.min(ip_v[pl.ds(t0, 16)])
        pos0 = pl.multiple_of(pos0, _PCHUNK)
        src = plane * NUM_TOK + t0
        dst = plane * NUM_TOK + pos0
        return src, dst

    def p_issue_gather(slot, t, src_hbm):
        src, _ = p_task(t)
        pltpu.async_copy(src_hbm.at[pl.ds(src, _PCHUNK)], spm.at[lslot, slot], sg2)

    def p_drain(sem):
        pltpu.make_async_copy(
            k_hbm.at[pl.ds(0, _PCHUNK)], spm.at[0, 0], sem).wait()

    @pl.when(wid >= _NSTREAM)
    def _():
        p_issue_gather(0, 0, k_hbm)
        p_issue_gather(1, 1, v_hbm)

    bq = wid // (_NW // B)
    seg = (wid % (_NW // B)) * _POS_SEG
    pltpu.sync_copy(pos_hbm.at[bq, pl.ds(seg, _POS_SEG)], posb)
    for j in range(_POS_VECS):
        vals = ip_v[pl.ds(j * 16, 16)]
        mask = (vals >= seg) & (vals < seg + _POS_SEG)
        plsc.store_scatter(posb, [vals - seg], vals, mask=mask)
    pltpu.sync_copy(posb, po_hbm.at[bq, pl.ds(seg, _POS_SEG)])

    @pl.when(wid < _NSTREAM)
    def _():
        def outer(i, carry):
            for slot in range(_NBUF):
                t = i * _NBUF + slot
                dst_hbm = ko_hbm if slot % 2 == 0 else vo_hbm

                @pl.when(t >= 2)
                def _():
                    s_drain(ss)

                s_drain(sg)
                pltpu.async_copy(buf.at[slot], dst_hbm.at[idx_v.at[slot]], ss)

                nslot = (slot + 2) % _NBUF
                nsrc = k_hbm if nslot % 2 == 0 else v_hbm

                @pl.when(t + 2 < _NTASK_S)
                def _():
                    s_issue_gather(nslot, t + 2, nsrc)
            return carry

        lax.fori_loop(0, _NTASK_S // _NBUF, outer, 0)
        s_drain(ss)
        s_drain(ss)

    @pl.when(wid >= _NSTREAM)
    def _():
        def outer(i, carry):
            for slot in range(_NBUF):
                t = i * _NBUF + slot
                dst_hbm = ko_hbm if slot % 2 == 0 else vo_hbm

                @pl.when(t >= 2)
                def _():
                    p_drain(ss2)

                p_drain(sg2)
                _, dst = p_task(t)
                pltpu.async_copy(
                    spm.at[lslot, slot], dst_hbm.at[pl.ds(dst, _PCHUNK)], ss2)

                nslot = (slot + 2) % _NBUF
                nsrc = k_hbm if nslot % 2 == 0 else v_hbm

                @pl.when(t + 2 < _NTASK_P)
                def _():
                    p_issue_gather(nslot, t + 2, nsrc)
            return carry

        lax.fori_loop(0, _NTASK_P // _NBUF, outer, 0)
        p_drain(ss2)
        p_drain(ss2)


@jax.jit
def _sc_update(input_pos, k_src, v_src, pos_src):
    n_rows = _BH * NUM_TOK
    run = pl.kernel(
        _sc_body,
        out_type=(
            jax.ShapeDtypeStruct((n_rows, D), jnp.float32),
            jax.ShapeDtypeStruct((n_rows, D), jnp.float32),
            jax.ShapeDtypeStruct((B, MAX_CACHE), jnp.int32),
        ),
        mesh=plsc.VectorSubcoreMesh(core_axis_name="c", subcore_axis_name="s"),
        compiler_params=pltpu.CompilerParams(needs_layout_passes=False),
        scratch_types=(
            pltpu.VMEM((NUM_TOK,), jnp.int32),
            pltpu.VMEM((_NBUF, _CHUNK), jnp.int32),
            pltpu.VMEM((_NBUF, _CHUNK, D), jnp.float32),
            pltpu.VMEM((_POS_SEG,), jnp.int32),
            pltpu.VMEM_SHARED((2, _NBUF, _PCHUNK, D), jnp.float32),
            pltpu.SemaphoreType.DMA,
            pltpu.SemaphoreType.DMA,
            pltpu.SemaphoreType.DMA,
            pltpu.SemaphoreType.DMA,
        ),
    )
    return run(input_pos, k_src, v_src, pos_src)


def kernel(input_pos, k_val, v_val, k_cache, v_cache, pos):
    del k_cache, v_cache
    k_src = k_val.reshape(_BH * NUM_TOK, D)
    v_src = v_val.reshape(_BH * NUM_TOK, D)
    pos_src = pos.reshape(B, MAX_CACHE)
    k_flat, v_flat, pos_flat = _sc_update(input_pos, k_src, v_src, pos_src)
    return (
        k_flat.reshape(B, H, NUM_TOK, D),
        v_flat.reshape(B, H, NUM_TOK, D),
        pos_flat.reshape(B, 1, MAX_CACHE),
    )
